# Initial kernel scaffold; baseline (speedup 1.0000x reference)
import functools

import jax
import jax.numpy as jnp
from jax import lax
from jax.experimental import pallas as pl
from jax.experimental.pallas import tpu as pltpu
from jax.experimental.pallas import tpu_sc as plsc

N_NODES = 10000
N_EDGES = 160000
D_FEAT = 256
D_HALF = D_FEAT // 2

NC, NS, LANES = 2, 16, 16
BATCH = 128
E_PAD = 163840
IDX_ROWS = E_PAD // BATCH
ROWS_PER_SUB = IDX_ROWS // NS
ACC_N = 10016
TRASH_ROW = N_NODES
ZSTRIPE = ACC_N // NS
WSTRIPE = N_NODES // NS


def _sc_aggregate(x_lo, x_hi, src2d, dst2d, zeros):
    mesh = plsc.VectorSubcoreMesh(core_axis_name="c", subcore_axis_name="s")

    @functools.partial(
        pl.kernel,
        mesh=mesh,
        out_type=[
            jax.ShapeDtypeStruct((N_NODES, D_HALF), jnp.float32),
            jax.ShapeDtypeStruct((N_NODES, D_HALF), jnp.float32),
            jax.ShapeDtypeStruct((N_NODES, LANES), jnp.float32),
        ],
        scratch_types=[
            pltpu.VMEM((ROWS_PER_SUB, BATCH), jnp.int32),
            pltpu.VMEM((ROWS_PER_SUB, BATCH), jnp.int32),
            pltpu.VMEM((BATCH, D_HALF), jnp.float32),
            pltpu.VMEM((BATCH, D_HALF), jnp.float32),
            pltpu.VMEM((BATCH, LANES), jnp.float32),
            pltpu.VMEM_SHARED((ACC_N, D_HALF), jnp.float32),
            pltpu.VMEM_SHARED((ACC_N, LANES), jnp.float32),
            pltpu.SemaphoreType.DMA,
            pltpu.SemaphoreType.DMA,
        ],
    )
    def sc_kernel(xlo_hbm, xhi_hbm, src_hbm, dst_hbm, zeros_hbm,
                  agglo_hbm, agghi_hbm, deg_hbm,
                  srcv, dstv, bufa, bufb, ones_v, acc, dacc, sema, semb):
        cid = lax.axis_index("c")
        sid = lax.axis_index("s")

        z0 = sid * ZSTRIPE
        pltpu.sync_copy(zeros_hbm.at[pl.ds(z0, ZSTRIPE)],
                        acc.at[pl.ds(z0, ZSTRIPE)])

        @pl.when(cid == 0)
        def _():
            pltpu.sync_copy(zeros_hbm.at[pl.ds(z0, ZSTRIPE), pl.ds(0, LANES)],
                            dacc.at[pl.ds(z0, ZSTRIPE)])

        @pl.loop(0, BATCH)
        def _(i):
            ones_v[i, :] = jnp.ones((LANES,), jnp.float32)

        i0 = sid * ROWS_PER_SUB
        pltpu.sync_copy(src_hbm.at[pl.ds(i0, ROWS_PER_SUB)], srcv)
        pltpu.sync_copy(dst_hbm.at[pl.ds(i0, ROWS_PER_SUB)], dstv)

        plsc.subcore_barrier()

        def run(table_hbm, do_deg):
            pltpu.async_copy(table_hbm.at[srcv.at[0]], bufa, sema)

            @pl.loop(0, ROWS_PER_SUB, step=2)
            def _(j):
                pltpu.make_async_copy(table_hbm.at[srcv.at[j]], bufa,
                                      sema).wait()
                pltpu.async_copy(table_hbm.at[srcv.at[j + 1]], bufb, semb)
                pltpu.sync_copy(bufa, acc.at[dstv.at[j]], add=True)
                if do_deg:
                    pltpu.sync_copy(ones_v, dacc.at[dstv.at[j]], add=True)

                pltpu.make_async_copy(table_hbm.at[srcv.at[j + 1]], bufb,
                                      semb).wait()

                @pl.when(j + 2 < ROWS_PER_SUB)
                def _():
                    pltpu.async_copy(table_hbm.at[srcv.at[j + 2]], bufa, sema)

                pltpu.sync_copy(bufb, acc.at[dstv.at[j + 1]], add=True)
                if do_deg:
                    pltpu.sync_copy(ones_v, dacc.at[dstv.at[j + 1]], add=True)

        @pl.when(cid == 0)
        def _():
            run(xlo_hbm, True)

        @pl.when(cid == 1)
        def _():
            run(xhi_hbm, False)

        plsc.subcore_barrier()

        w0 = sid * WSTRIPE

        @pl.when(cid == 0)
        def _():
            pltpu.sync_copy(acc.at[pl.ds(w0, WSTRIPE)],
                            agglo_hbm.at[pl.ds(w0, WSTRIPE)])
            pltpu.sync_copy(dacc.at[pl.ds(w0, WSTRIPE)],
                            deg_hbm.at[pl.ds(w0, WSTRIPE)])

        @pl.when(cid == 1)
        def _():
            pltpu.sync_copy(acc.at[pl.ds(w0, WSTRIPE)],
                            agghi_hbm.at[pl.ds(w0, WSTRIPE)])

    return sc_kernel(x_lo, x_hi, src2d, dst2d, zeros)


def _divide_body(lo_ref, hi_ref, deg_ref, o_ref):
    d = deg_ref[:, 0:1]
    d = jnp.where(d == 0.0, 1.0, d)
    o_ref[:, :D_HALF] = lo_ref[...] / d
    o_ref[:, D_HALF:] = hi_ref[...] / d


def _tc_divide(agg_lo, agg_hi, deg):
    blk = 1000
    grid = N_NODES // blk
    return pl.pallas_call(
        _divide_body,
        grid=(grid,),
        in_specs=[
            pl.BlockSpec((blk, D_HALF), lambda i: (i, 0)),
            pl.BlockSpec((blk, D_HALF), lambda i: (i, 0)),
            pl.BlockSpec((blk, LANES), lambda i: (i, 0)),
        ],
        out_specs=pl.BlockSpec((blk, D_FEAT), lambda i: (i, 0)),
        out_shape=jax.ShapeDtypeStruct((N_NODES, D_FEAT), jnp.float32),
    )(agg_lo, agg_hi, deg)


@jax.jit
def kernel(x, edge_index):
    dst = edge_index[0].astype(jnp.int32)
    src = edge_index[1].astype(jnp.int32)
    pad = E_PAD - N_EDGES
    src_p = jnp.concatenate([src, jnp.zeros((pad,), jnp.int32)])
    dst_p = jnp.concatenate([dst, jnp.full((pad,), TRASH_ROW, jnp.int32)])
    src2d = src_p.reshape(IDX_ROWS, BATCH)
    dst2d = dst_p.reshape(IDX_ROWS, BATCH)
    x_lo = x[:, :D_HALF]
    x_hi = x[:, D_HALF:]
    zeros = jnp.zeros((ACC_N, D_HALF), jnp.float32)
    agg_lo, agg_hi, deg = _sc_aggregate(x_lo, x_hi, src2d, dst2d, zeros)
    return _tc_divide(agg_lo, agg_hi, deg)


# speedup vs baseline: 2.4441x; 2.4441x over previous
import functools

import jax
import jax.numpy as jnp
from jax import lax
from jax.experimental import pallas as pl
from jax.experimental.pallas import tpu as pltpu
from jax.experimental.pallas import tpu_sc as plsc

N_NODES = 10000
N_EDGES = 160000
D_FEAT = 256
D_HALF = D_FEAT // 2

NC, NS = 2, 16
NW = NC * NS
BATCH = 128
E_PAD = 163840
PER_SUB = E_PAD // NS
PER_WORKER = E_PAD // NW
ACC_N = 10112
TRASH_ROW = N_NODES
WSTRIPE = 632
WLAST = N_NODES - WSTRIPE
WCHUNK = 160
WOFFS = (0, 160, 320, 472)


def _writeback(acc, wbuf, out_hbm, sid, out_base):
    w0 = jnp.minimum(sid * WSTRIPE, WLAST)
    for off in WOFFS:
        pltpu.sync_copy(acc.at[pl.ds(w0 + off, WCHUNK)], wbuf)
        pltpu.sync_copy(wbuf, out_hbm.at[pl.ds(out_base + w0 + off, WCHUNK)])


def _sc_features(table, src2, dst1, zeros):
    mesh = plsc.VectorSubcoreMesh(core_axis_name="c", subcore_axis_name="s")

    @functools.partial(
        pl.kernel,
        mesh=mesh,
        out_type=jax.ShapeDtypeStruct((NC * N_NODES, D_HALF), jnp.float32),
        scratch_types=[
            pltpu.VMEM((BATCH,), jnp.int32),
            pltpu.VMEM((BATCH,), jnp.int32),
            pltpu.VMEM((BATCH, D_HALF), jnp.float32),
            pltpu.VMEM((WCHUNK, D_HALF), jnp.float32),
            pltpu.VMEM_SHARED((ACC_N, D_HALF), jnp.float32),
            pltpu.SemaphoreType.DMA,
        ],
    )
    def sc_kernel(table_hbm, src_hbm, dst_hbm, zeros_hbm, agg_hbm,
                  src_v, dst_v, rows_v, wbuf, acc, sem):
        cid = lax.axis_index("c")
        sid = lax.axis_index("s")

        @pl.when(sid == 0)
        def _():
            pltpu.sync_copy(zeros_hbm, acc)

        plsc.subcore_barrier()

        sbase = cid * E_PAD + sid * PER_SUB
        dbase = sid * PER_SUB

        @pl.loop(0, PER_SUB // BATCH)
        def _(j):
            pltpu.sync_copy(src_hbm.at[pl.ds(sbase + j * BATCH, BATCH)],
                            src_v)
            pltpu.sync_copy(dst_hbm.at[pl.ds(dbase + j * BATCH, BATCH)],
                            dst_v)
            pltpu.async_copy(table_hbm.at[src_v], rows_v, sem).wait()
            pltpu.sync_copy(rows_v, acc.at[dst_v], add=True)

        plsc.subcore_barrier()
        _writeback(acc, wbuf, agg_hbm, sid, cid * N_NODES)

    return sc_kernel(table, src2, dst1, zeros)


def _sc_degree(dst1, zeros, ones):
    mesh = plsc.VectorSubcoreMesh(core_axis_name="c", subcore_axis_name="s")

    @functools.partial(
        pl.kernel,
        mesh=mesh,
        out_type=jax.ShapeDtypeStruct((NC * N_NODES, D_HALF), jnp.float32),
        scratch_types=[
            pltpu.VMEM((BATCH,), jnp.int32),
            pltpu.VMEM((BATCH, D_HALF), jnp.float32),
            pltpu.VMEM((WCHUNK, D_HALF), jnp.float32),
            pltpu.VMEM_SHARED((ACC_N, D_HALF), jnp.float32),
        ],
    )
    def sc_kernel(dst_hbm, zeros_hbm, ones_hbm, deg_hbm,
                  dst_v, ones_v, wbuf, dacc):
        cid = lax.axis_index("c")
        sid = lax.axis_index("s")

        @pl.when(sid == 0)
        def _():
            pltpu.sync_copy(zeros_hbm, dacc)

        pltpu.sync_copy(ones_hbm, ones_v)
        plsc.subcore_barrier()

        base = (sid * NC + cid) * PER_WORKER

        @pl.loop(0, PER_WORKER // BATCH)
        def _(j):
            pltpu.sync_copy(dst_hbm.at[pl.ds(base + j * BATCH, BATCH)],
                            dst_v)
            pltpu.sync_copy(ones_v, dacc.at[dst_v], add=True)

        plsc.subcore_barrier()
        _writeback(dacc, wbuf, deg_hbm, sid, cid * N_NODES)

    return sc_kernel(dst1, zeros, ones)


def _divide_body(lo_ref, hi_ref, d0_ref, d1_ref, o_ref):
    d = d0_ref[:, 0:1] + d1_ref[:, 0:1]
    d = jnp.where(d == 0.0, 1.0, d)
    o_ref[:, :D_HALF] = lo_ref[...] / d
    o_ref[:, D_HALF:] = hi_ref[...] / d


def _tc_divide(agg, deg):
    blk = 1000
    nblk = N_NODES // blk
    return pl.pallas_call(
        _divide_body,
        grid=(nblk,),
        in_specs=[
            pl.BlockSpec((blk, D_HALF), lambda i: (i, 0)),
            pl.BlockSpec((blk, D_HALF), lambda i: (i + nblk, 0)),
            pl.BlockSpec((blk, D_HALF), lambda i: (i, 0)),
            pl.BlockSpec((blk, D_HALF), lambda i: (i + nblk, 0)),
        ],
        out_specs=pl.BlockSpec((blk, D_FEAT), lambda i: (i, 0)),
        out_shape=jax.ShapeDtypeStruct((N_NODES, D_FEAT), jnp.float32),
    )(agg, agg, deg, deg)


@jax.jit
def kernel(x, edge_index):
    dst = edge_index[0].astype(jnp.int32)
    src = edge_index[1].astype(jnp.int32)
    pad = E_PAD - N_EDGES
    src_p = jnp.concatenate([src, jnp.zeros((pad,), jnp.int32)])
    dst_p = jnp.concatenate([dst, jnp.full((pad,), TRASH_ROW, jnp.int32)])
    src2 = jnp.concatenate([src_p, src_p + N_NODES])
    table = jnp.concatenate([x[:, :D_HALF], x[:, D_HALF:]])
    zeros = jnp.zeros((ACC_N, D_HALF), jnp.float32)
    ones = jnp.ones((BATCH, D_HALF), jnp.float32)
    agg = _sc_features(table, src2, dst_p, zeros)
    deg = _sc_degree(dst_p, zeros, ones)
    return _tc_divide(agg, deg)


# speedup vs baseline: 2.9969x; 1.2262x over previous
import functools

import jax
import jax.numpy as jnp
from jax import lax
from jax.experimental import pallas as pl
from jax.experimental.pallas import tpu as pltpu
from jax.experimental.pallas import tpu_sc as plsc

N_NODES = 10000
N_EDGES = 160000
D_FEAT = 256
D_HALF = D_FEAT // 2

NC, NS = 2, 16
NW = NC * NS
BATCH = 128
E_PAD = 163840
PER_SUB = E_PAD // NS
PER_WORKER = E_PAD // NW
ACC_N = 10112
TRASH_ROW = N_NODES
WSTRIPE = 632
WLAST = N_NODES - WSTRIPE
WCHUNK = 80
WOFFS = (0, 80, 160, 240, 320, 400, 480, 552)
CHUNK_B = 16
N_CHUNKS = PER_SUB // (CHUNK_B * BATCH)


def _writeback(acc, wbuf, out_hbm, sid, out_base):
    w0 = jnp.minimum(sid * WSTRIPE, WLAST)
    for off in WOFFS:
        pltpu.sync_copy(acc.at[pl.ds(w0 + off, WCHUNK)], wbuf)
        pltpu.sync_copy(wbuf, out_hbm.at[pl.ds(out_base + w0 + off, WCHUNK)])


def _sc_features(table, src2, dst1, zeros):
    mesh = plsc.VectorSubcoreMesh(core_axis_name="c", subcore_axis_name="s")

    @functools.partial(
        pl.kernel,
        mesh=mesh,
        out_type=jax.ShapeDtypeStruct((NC * N_NODES, D_HALF), jnp.float32),
        scratch_types=[
            pltpu.VMEM((CHUNK_B, BATCH), jnp.int32),
            pltpu.VMEM((CHUNK_B, BATCH), jnp.int32),
            pltpu.VMEM((BATCH, D_HALF), jnp.float32),
            pltpu.VMEM((BATCH, D_HALF), jnp.float32),
            pltpu.VMEM((WCHUNK, D_HALF), jnp.float32),
            pltpu.VMEM_SHARED((ACC_N, D_HALF), jnp.float32),
            pltpu.SemaphoreType.DMA,
            pltpu.SemaphoreType.DMA,
        ],
    )
    def sc_kernel(table_hbm, src_hbm, dst_hbm, zeros_hbm, agg_hbm,
                  src_c, dst_c, rows_a, rows_b, wbuf, acc, sem_g, sem_s):
        cid = lax.axis_index("c")
        sid = lax.axis_index("s")

        @pl.when(sid == 0)
        def _():
            pltpu.sync_copy(zeros_hbm, acc)

        plsc.subcore_barrier()

        srow = cid * (E_PAD // BATCH) + sid * (PER_SUB // BATCH)
        drow = sid * (PER_SUB // BATCH)
        rows = (rows_a, rows_b)

        @pl.loop(0, N_CHUNKS)
        def _(c):
            pltpu.sync_copy(src_hbm.at[pl.ds(srow + c * CHUNK_B, CHUNK_B)],
                            src_c)
            pltpu.sync_copy(dst_hbm.at[pl.ds(drow + c * CHUNK_B, CHUNK_B)],
                            dst_c)
            pltpu.async_copy(table_hbm.at[src_c.at[0]], rows[0], sem_g)
            for j in range(CHUNK_B):
                cur, nxt = rows[j % 2], rows[(j + 1) % 2]
                pltpu.make_async_copy(table_hbm.at[src_c.at[j]], cur,
                                      sem_g).wait()
                if j + 1 < CHUNK_B:
                    if j >= 1:
                        pltpu.make_async_copy(nxt, acc.at[dst_c.at[j - 1]],
                                              sem_s).wait()
                    pltpu.async_copy(table_hbm.at[src_c.at[j + 1]], nxt,
                                     sem_g)
                pltpu.async_copy(cur, acc.at[dst_c.at[j]], sem_s, add=True)
            pltpu.make_async_copy(rows[(CHUNK_B - 2) % 2],
                                  acc.at[dst_c.at[CHUNK_B - 2]],
                                  sem_s).wait()
            pltpu.make_async_copy(rows[(CHUNK_B - 1) % 2],
                                  acc.at[dst_c.at[CHUNK_B - 1]],
                                  sem_s).wait()

        plsc.subcore_barrier()
        _writeback(acc, wbuf, agg_hbm, sid, cid * N_NODES)

    return sc_kernel(table, src2, dst1, zeros)


def _sc_degree(dst1, zeros, ones):
    mesh = plsc.VectorSubcoreMesh(core_axis_name="c", subcore_axis_name="s")

    @functools.partial(
        pl.kernel,
        mesh=mesh,
        out_type=jax.ShapeDtypeStruct((NC * N_NODES, D_HALF), jnp.float32),
        scratch_types=[
            pltpu.VMEM((BATCH,), jnp.int32),
            pltpu.VMEM((BATCH, D_HALF), jnp.float32),
            pltpu.VMEM((WCHUNK, D_HALF), jnp.float32),
            pltpu.VMEM_SHARED((ACC_N, D_HALF), jnp.float32),
        ],
    )
    def sc_kernel(dst_hbm, zeros_hbm, ones_hbm, deg_hbm,
                  dst_v, ones_v, wbuf, dacc):
        cid = lax.axis_index("c")
        sid = lax.axis_index("s")

        @pl.when(sid == 0)
        def _():
            pltpu.sync_copy(zeros_hbm, dacc)

        pltpu.sync_copy(ones_hbm, ones_v)
        plsc.subcore_barrier()

        base = (sid * NC + cid) * (PER_WORKER // BATCH)

        @pl.loop(0, PER_WORKER // BATCH)
        def _(j):
            pltpu.sync_copy(dst_hbm.at[base + j], dst_v)
            pltpu.sync_copy(ones_v, dacc.at[dst_v], add=True)

        plsc.subcore_barrier()
        _writeback(dacc, wbuf, deg_hbm, sid, cid * N_NODES)

    return sc_kernel(dst1, zeros, ones)


def _divide_body(lo_ref, hi_ref, d0_ref, d1_ref, o_ref):
    d = d0_ref[:, 0:1] + d1_ref[:, 0:1]
    d = jnp.where(d == 0.0, 1.0, d)
    o_ref[:, :D_HALF] = lo_ref[...] / d
    o_ref[:, D_HALF:] = hi_ref[...] / d


def _tc_divide(agg, deg):
    blk = 1000
    nblk = N_NODES // blk
    return pl.pallas_call(
        _divide_body,
        grid=(nblk,),
        in_specs=[
            pl.BlockSpec((blk, D_HALF), lambda i: (i, 0)),
            pl.BlockSpec((blk, D_HALF), lambda i: (i + nblk, 0)),
            pl.BlockSpec((blk, D_HALF), lambda i: (i, 0)),
            pl.BlockSpec((blk, D_HALF), lambda i: (i + nblk, 0)),
        ],
        out_specs=pl.BlockSpec((blk, D_FEAT), lambda i: (i, 0)),
        out_shape=jax.ShapeDtypeStruct((N_NODES, D_FEAT), jnp.float32),
    )(agg, agg, deg, deg)


@jax.jit
def kernel(x, edge_index):
    dst = edge_index[0].astype(jnp.int32)
    src = edge_index[1].astype(jnp.int32)
    pad = E_PAD - N_EDGES
    src_p = jnp.concatenate([src, jnp.zeros((pad,), jnp.int32)])
    dst_p = jnp.concatenate([dst, jnp.full((pad,), TRASH_ROW, jnp.int32)])
    src2 = jnp.concatenate([src_p, src_p + N_NODES])
    src2 = src2.reshape(NC * E_PAD // BATCH, BATCH)
    dst_p = dst_p.reshape(E_PAD // BATCH, BATCH)
    table = jnp.concatenate([x[:, :D_HALF], x[:, D_HALF:]])
    zeros = jnp.zeros((ACC_N, D_HALF), jnp.float32)
    ones = jnp.ones((BATCH, D_HALF), jnp.float32)
    agg = _sc_features(table, src2, dst_p, zeros)
    deg = _sc_degree(dst_p, zeros, ones)
    return _tc_divide(agg, deg)


# speedup vs baseline: 3.1132x; 1.0388x over previous
import functools

import jax
import jax.numpy as jnp
from jax import lax
from jax.experimental import pallas as pl
from jax.experimental.pallas import tpu as pltpu
from jax.experimental.pallas import tpu_sc as plsc

N_NODES = 10000
N_EDGES = 160000
D_FEAT = 256
D_HALF = D_FEAT // 2

NC, NS = 2, 16
NW = NC * NS
BATCH = 64
DBATCH = 128
E_PAD = 163840
PER_SUB = E_PAD // NS
PER_WORKER = E_PAD // NW
ACC_N = 10112
TRASH_ROW = N_NODES
WSTRIPE = 632
WLAST = N_NODES - WSTRIPE
WCHUNK = 40
WOFFS = tuple(range(0, 600, 40)) + (592,)
NBUF = 5
CHUNK_B = 16
N_CHUNKS = PER_SUB // (CHUNK_B * BATCH)


def _writeback(acc, wbuf, out_hbm, sid, out_base):
    w0 = jnp.minimum(sid * WSTRIPE, WLAST)
    for off in WOFFS:
        pltpu.sync_copy(acc.at[pl.ds(w0 + off, WCHUNK)], wbuf)
        pltpu.sync_copy(wbuf, out_hbm.at[pl.ds(out_base + w0 + off, WCHUNK)])


def _sc_features(table, src2, dst1, zeros):
    mesh = plsc.VectorSubcoreMesh(core_axis_name="c", subcore_axis_name="s")

    @functools.partial(
        pl.kernel,
        mesh=mesh,
        out_type=jax.ShapeDtypeStruct((NC * N_NODES, D_HALF), jnp.float32),
        scratch_types=[
            pltpu.VMEM((CHUNK_B, BATCH), jnp.int32),
            pltpu.VMEM((CHUNK_B, BATCH), jnp.int32),
        ] + [
            pltpu.VMEM((BATCH, D_HALF), jnp.float32)
            for _ in range(NBUF)
        ] + [
            pltpu.VMEM((WCHUNK, D_HALF), jnp.float32),
            pltpu.VMEM_SHARED((ACC_N, D_HALF), jnp.float32),
            pltpu.SemaphoreType.DMA,
            pltpu.SemaphoreType.DMA,
        ],
    )
    def sc_kernel(table_hbm, src_hbm, dst_hbm, zeros_hbm, agg_hbm,
                  src_c, dst_c, *rest):
        bufs = rest[:NBUF]
        wbuf, acc, sem_g, sem_s = rest[NBUF:]
        cid = lax.axis_index("c")
        sid = lax.axis_index("s")

        @pl.when(sid == 0)
        def _():
            pltpu.sync_copy(zeros_hbm, acc)

        plsc.subcore_barrier()

        srow = cid * (E_PAD // BATCH) + sid * (PER_SUB // BATCH)
        drow = sid * (PER_SUB // BATCH)

        @pl.loop(0, N_CHUNKS)
        def _(c):
            pltpu.sync_copy(src_hbm.at[pl.ds(srow + c * CHUNK_B, CHUNK_B)],
                            src_c)
            pltpu.sync_copy(dst_hbm.at[pl.ds(drow + c * CHUNK_B, CHUNK_B)],
                            dst_c)
            for p in range(NBUF - 1):
                pltpu.async_copy(table_hbm.at[src_c.at[p]], bufs[p], sem_g)
            for j in range(CHUNK_B):
                cur = bufs[j % NBUF]
                pltpu.make_async_copy(table_hbm.at[src_c.at[j]], cur,
                                      sem_g).wait()
                nj = j + NBUF - 1
                if nj < CHUNK_B:
                    if j >= 1:
                        pltpu.make_async_copy(bufs[(j - 1) % NBUF],
                                              acc.at[dst_c.at[j - 1]],
                                              sem_s).wait()
                    pltpu.async_copy(table_hbm.at[src_c.at[nj]],
                                     bufs[nj % NBUF], sem_g)
                pltpu.async_copy(cur, acc.at[dst_c.at[j]], sem_s, add=True)
            for r in range(max(0, CHUNK_B - NBUF), CHUNK_B):
                pltpu.make_async_copy(bufs[r % NBUF],
                                      acc.at[dst_c.at[r]], sem_s).wait()

        plsc.subcore_barrier()
        _writeback(acc, wbuf, agg_hbm, sid, cid * N_NODES)

    return sc_kernel(table, src2, dst1, zeros)


def _sc_degree(dst1, zeros, ones):
    mesh = plsc.VectorSubcoreMesh(core_axis_name="c", subcore_axis_name="s")

    @functools.partial(
        pl.kernel,
        mesh=mesh,
        out_type=jax.ShapeDtypeStruct((NC * N_NODES, D_HALF), jnp.float32),
        scratch_types=[
            pltpu.VMEM((DBATCH,), jnp.int32),
            pltpu.VMEM((DBATCH, D_HALF), jnp.float32),
            pltpu.VMEM((WCHUNK, D_HALF), jnp.float32),
            pltpu.VMEM_SHARED((ACC_N, D_HALF), jnp.float32),
        ],
    )
    def sc_kernel(dst_hbm, zeros_hbm, ones_hbm, deg_hbm,
                  dst_v, ones_v, wbuf, dacc):
        cid = lax.axis_index("c")
        sid = lax.axis_index("s")

        @pl.when(sid == 0)
        def _():
            pltpu.sync_copy(zeros_hbm, dacc)

        pltpu.sync_copy(ones_hbm, ones_v)
        plsc.subcore_barrier()

        base = (sid * NC + cid) * (PER_WORKER // DBATCH)

        @pl.loop(0, PER_WORKER // DBATCH)
        def _(j):
            pltpu.sync_copy(dst_hbm.at[base + j], dst_v)
            pltpu.sync_copy(ones_v, dacc.at[dst_v], add=True)

        plsc.subcore_barrier()
        _writeback(dacc, wbuf, deg_hbm, sid, cid * N_NODES)

    return sc_kernel(dst1, zeros, ones)


def _divide_body(lo_ref, hi_ref, d0_ref, d1_ref, o_ref):
    d = d0_ref[:, 0:1] + d1_ref[:, 0:1]
    d = jnp.where(d == 0.0, 1.0, d)
    o_ref[:, :D_HALF] = lo_ref[...] / d
    o_ref[:, D_HALF:] = hi_ref[...] / d


def _tc_divide(agg, deg):
    blk = 1000
    nblk = N_NODES // blk
    return pl.pallas_call(
        _divide_body,
        grid=(nblk,),
        in_specs=[
            pl.BlockSpec((blk, D_HALF), lambda i: (i, 0)),
            pl.BlockSpec((blk, D_HALF), lambda i: (i + nblk, 0)),
            pl.BlockSpec((blk, D_HALF), lambda i: (i, 0)),
            pl.BlockSpec((blk, D_HALF), lambda i: (i + nblk, 0)),
        ],
        out_specs=pl.BlockSpec((blk, D_FEAT), lambda i: (i, 0)),
        out_shape=jax.ShapeDtypeStruct((N_NODES, D_FEAT), jnp.float32),
    )(agg, agg, deg, deg)


@jax.jit
def kernel(x, edge_index):
    dst = edge_index[0].astype(jnp.int32)
    src = edge_index[1].astype(jnp.int32)
    pad = E_PAD - N_EDGES
    src_p = jnp.concatenate([src, jnp.zeros((pad,), jnp.int32)])
    dst_p = jnp.concatenate([dst, jnp.full((pad,), TRASH_ROW, jnp.int32)])
    src2 = jnp.concatenate([src_p, src_p + N_NODES])
    src2 = src2.reshape(NC * E_PAD // BATCH, BATCH)
    dst64 = dst_p.reshape(E_PAD // BATCH, BATCH)
    dst128 = dst_p.reshape(E_PAD // DBATCH, DBATCH)
    table = jnp.concatenate([x[:, :D_HALF], x[:, D_HALF:]])
    zeros = jnp.zeros((ACC_N, D_HALF), jnp.float32)
    ones = jnp.ones((DBATCH, D_HALF), jnp.float32)
    agg = _sc_features(table, src2, dst64, zeros)
    deg = _sc_degree(dst128, zeros, ones)
    return _tc_divide(agg, deg)
